# baseline (speedup 1.0000x reference)
import jax
import jax.numpy as jnp
from jax import lax
from jax.experimental import pallas as pl
from jax.experimental.pallas import tpu as pltpu
from jax.experimental.pallas import tpu_sc as plsc

D = 128
B = 16384
S = 5
NCOL = 2 + S

NC = 2
NSUB = 16
NW = NC * NSUB
ROWS_PER_W = B // NW
CHUNK = 128
NBLK = ROWS_PER_W // CHUNK
NHALF = 2
NUNIT = NCOL * NHALF
NBIG = 3


def _sg_body(iw_hbm, ow_hbm, nst_hbm, in_tab, out_tab, out_hbm,
             idx_bufs, big, i_sem, g_sem, w_sem):
    wid = lax.axis_index("s") * NC + lax.axis_index("c")
    wb = wid * ROWS_PER_W

    for b in range(NBLK):
        sl = pl.ds(wb + b * CHUNK, CHUNK)
        pltpu.async_copy(iw_hbm.at[sl], idx_bufs[b].at[0], i_sem)
        pltpu.async_copy(ow_hbm.at[sl], idx_bufs[b].at[1], i_sem)
        pltpu.async_copy(nst_hbm.at[:, sl], idx_bufs[b].at[pl.ds(2, S)], i_sem)
    idx_waited = 0

    def need_idx(upto):
        nonlocal idx_waited
        while idx_waited < upto:
            pltpu.make_async_copy(
                iw_hbm.at[pl.ds(wb, CHUNK)], idx_bufs[idx_waited].at[0], i_sem
            ).wait()
            pltpu.make_async_copy(
                ow_hbm.at[pl.ds(wb, CHUNK)], idx_bufs[idx_waited].at[1], i_sem
            ).wait()
            pltpu.make_async_copy(
                nst_hbm.at[:, pl.ds(wb, CHUNK)],
                idx_bufs[idx_waited].at[pl.ds(2, S)], i_sem
            ).wait()
            idx_waited += 1

    def drain_gathers(p):
        pltpu.make_async_copy(
            out_tab.at[pl.ds(0, NHALF * CHUNK)], big[p], g_sem
        ).wait()

    def wait_write():
        pltpu.make_async_copy(
            big[0], out_hbm.at[0, pl.ds(wb, NHALF * CHUNK)], w_sem
        ).wait()

    prev = None
    for m in range(NUNIT):
        c, h, p = m // NHALF, m % NHALF, m % NBIG
        if m >= NBIG:
            wait_write()
        tab = in_tab if c == 0 else out_tab
        need_idx((h + 1) * NHALF)
        for j in range(NHALF):
            pltpu.async_copy(
                tab.at[idx_bufs[h * NHALF + j].at[c]],
                big[p].at[pl.ds(j * CHUNK, CHUNK)],
                g_sem,
            )
        if prev is not None:
            pc, ph = prev
            drain_gathers((m - 1) % NBIG)
            pltpu.async_copy(
                big[(m - 1) % NBIG],
                out_hbm.at[pc, pl.ds(wb + ph * NHALF * CHUNK, NHALF * CHUNK)],
                w_sem,
            )
        prev = (c, h)
    pc, ph = prev
    drain_gathers((NUNIT - 1) % NBIG)
    pltpu.async_copy(
        big[(NUNIT - 1) % NBIG],
        out_hbm.at[pc, pl.ds(wb + ph * NHALF * CHUNK, NHALF * CHUNK)],
        w_sem,
    )
    for _ in range(NBIG):
        wait_write()


def kernel(input_words, output_words, neg_samples, input_embed, output_embed):
    iw = input_words.astype(jnp.int32)
    ow = output_words.astype(jnp.int32)
    nst = neg_samples.T.astype(jnp.int32)
    mesh = plsc.VectorSubcoreMesh(core_axis_name="c", subcore_axis_name="s")
    f = pl.kernel(
        _sg_body,
        out_type=jax.ShapeDtypeStruct((NCOL, B, D), jnp.float32),
        mesh=mesh,
        compiler_params=pltpu.CompilerParams(skip_device_barrier=True),
        scratch_types=[
            [pltpu.VMEM((NCOL, CHUNK), jnp.int32) for _ in range(NBLK)],
            [pltpu.VMEM((NHALF * CHUNK, D), jnp.float32) for _ in range(NBIG)],
            pltpu.SemaphoreType.DMA,
            pltpu.SemaphoreType.DMA,
            pltpu.SemaphoreType.DMA,
        ],
    )
    out = f(iw, ow, nst, input_embed, output_embed)
    return jnp.transpose(out, (1, 0, 2))


# speedup vs baseline: 1.0118x; 1.0118x over previous
import jax
import jax.numpy as jnp
from jax import lax
from jax.experimental import pallas as pl
from jax.experimental.pallas import tpu as pltpu
from jax.experimental.pallas import tpu_sc as plsc

D = 128
B = 16384
S = 5
NCOL = 2 + S

NC = 2
NSUB = 16
NW = NC * NSUB
ROWS_PER_W = B // NW
CHUNK = 128
NBLK = ROWS_PER_W // CHUNK
NHALF = 2
NUNIT = NCOL * NHALF
NBIG = 3


def _sg_body(idx_hbm, in_tab, out_tab, out_hbm, idx_bufs, big, i_sem, g_sem, w_sem):
    wid = lax.axis_index("s") * NC + lax.axis_index("c")
    wb = wid * ROWS_PER_W

    for b in range(NBLK):
        pltpu.async_copy(
            idx_hbm.at[:, pl.ds(wb + b * CHUNK, CHUNK)], idx_bufs[b], i_sem
        )
    idx_waited = 0

    def need_idx(upto):
        nonlocal idx_waited
        while idx_waited < upto:
            pltpu.make_async_copy(
                idx_hbm.at[:, pl.ds(wb, CHUNK)], idx_bufs[idx_waited], i_sem
            ).wait()
            idx_waited += 1

    def drain_gathers(p):
        pltpu.make_async_copy(
            out_tab.at[pl.ds(0, NHALF * CHUNK)], big[p], g_sem
        ).wait()

    def wait_write():
        pltpu.make_async_copy(
            big[0], out_hbm.at[0, pl.ds(wb, NHALF * CHUNK)], w_sem
        ).wait()

    prev = None
    for m in range(NUNIT):
        c, h, p = m // NHALF, m % NHALF, m % NBIG
        if m >= NBIG:
            wait_write()
        tab = in_tab if c == 0 else out_tab
        need_idx((h + 1) * NHALF)
        for j in range(NHALF):
            pltpu.async_copy(
                tab.at[idx_bufs[h * NHALF + j].at[c]],
                big[p].at[pl.ds(j * CHUNK, CHUNK)],
                g_sem,
            )
        if prev is not None:
            pc, ph = prev
            drain_gathers((m - 1) % NBIG)
            pltpu.async_copy(
                big[(m - 1) % NBIG],
                out_hbm.at[pc, pl.ds(wb + ph * NHALF * CHUNK, NHALF * CHUNK)],
                w_sem,
            )
        prev = (c, h)
    pc, ph = prev
    drain_gathers((NUNIT - 1) % NBIG)
    pltpu.async_copy(
        big[(NUNIT - 1) % NBIG],
        out_hbm.at[pc, pl.ds(wb + ph * NHALF * CHUNK, NHALF * CHUNK)],
        w_sem,
    )
    for _ in range(NBIG):
        wait_write()


def kernel(input_words, output_words, neg_samples, input_embed, output_embed):
    idx_all = jnp.concatenate(
        [input_words[None, :], output_words[None, :], neg_samples.T], axis=0
    ).astype(jnp.int32)
    mesh = plsc.VectorSubcoreMesh(core_axis_name="c", subcore_axis_name="s")
    f = pl.kernel(
        _sg_body,
        out_type=jax.ShapeDtypeStruct((NCOL, B, D), jnp.float32),
        mesh=mesh,
        compiler_params=pltpu.CompilerParams(skip_device_barrier=True),
        scratch_types=[
            [pltpu.VMEM((NCOL, CHUNK), jnp.int32) for _ in range(NBLK)],
            [pltpu.VMEM((NHALF * CHUNK, D), jnp.float32) for _ in range(NBIG)],
            pltpu.SemaphoreType.DMA,
            pltpu.SemaphoreType.DMA,
            pltpu.SemaphoreType.DMA,
        ],
    )
    out = f(idx_all, input_embed, output_embed)
    return jnp.transpose(out, (1, 0, 2))


# speedup vs baseline: 1.0124x; 1.0006x over previous
import jax
import jax.numpy as jnp
from jax import lax
from jax.experimental import pallas as pl
from jax.experimental.pallas import tpu as pltpu
from jax.experimental.pallas import tpu_sc as plsc

D = 128
B = 16384
S = 5
NCOL = 2 + S

NC = 2
NSUB = 16
NW = NC * NSUB
ROWS_PER_W = B // NW
CHUNK = 128
NBLK = ROWS_PER_W // CHUNK
NHALF = 2
NUNIT = NCOL * NHALF
NBIG = 3


def _sg_body(idx_hbm, in_tab, out_tab, out_hbm, idx_bufs, big, i_sem, g_sem, w_sem):
    wid = lax.axis_index("s") * NC + lax.axis_index("c")
    wb = wid * ROWS_PER_W

    for b in range(NBLK):
        pltpu.async_copy(
            idx_hbm.at[:, pl.ds(wb + b * CHUNK, CHUNK)], idx_bufs[b], i_sem
        )
    idx_waited = 0

    def need_idx(upto):
        nonlocal idx_waited
        while idx_waited < upto:
            pltpu.make_async_copy(
                idx_hbm.at[:, pl.ds(wb, CHUNK)], idx_bufs[idx_waited], i_sem
            ).wait()
            idx_waited += 1

    def drain_gathers(p):
        pltpu.make_async_copy(
            out_tab.at[pl.ds(0, NHALF * CHUNK)], big[p], g_sem
        ).wait()

    def wait_write():
        pltpu.make_async_copy(
            big[0], out_hbm.at[0, pl.ds(wb, NHALF * CHUNK)], w_sem
        ).wait()

    prev = None
    for m in range(NUNIT):
        c, h, p = m // NHALF, m % NHALF, m % NBIG
        if m >= NBIG:
            wait_write()
        tab = in_tab if c == 0 else out_tab
        need_idx((h + 1) * NHALF)
        for j in range(NHALF):
            pltpu.async_copy(
                tab.at[idx_bufs[h * NHALF + j].at[c]],
                big[p].at[pl.ds(j * CHUNK, CHUNK)],
                g_sem,
            )
        if prev is not None:
            pc, ph = prev
            drain_gathers((m - 1) % NBIG)
            pltpu.async_copy(
                big[(m - 1) % NBIG],
                out_hbm.at[pc, pl.ds(wb + ph * NHALF * CHUNK, NHALF * CHUNK)],
                w_sem,
            )
        prev = (c, h)
    pc, ph = prev
    drain_gathers((NUNIT - 1) % NBIG)
    pltpu.async_copy(
        big[(NUNIT - 1) % NBIG],
        out_hbm.at[pc, pl.ds(wb + ph * NHALF * CHUNK, NHALF * CHUNK)],
        w_sem,
    )
    for _ in range(NBIG):
        wait_write()


def kernel(input_words, output_words, neg_samples, input_embed, output_embed):
    idx_all = jnp.concatenate(
        [input_words[None, :], output_words[None, :], neg_samples.T], axis=0
    ).astype(jnp.int32)
    mesh = plsc.VectorSubcoreMesh(core_axis_name="c", subcore_axis_name="s")
    f = pl.kernel(
        _sg_body,
        out_type=jax.ShapeDtypeStruct((NCOL, B, D), jnp.float32),
        mesh=mesh,
        scratch_types=[
            [pltpu.VMEM((NCOL, CHUNK), jnp.int32) for _ in range(NBLK)],
            [pltpu.VMEM((NHALF * CHUNK, D), jnp.float32) for _ in range(NBIG)],
            pltpu.SemaphoreType.DMA,
            pltpu.SemaphoreType.DMA,
            pltpu.SemaphoreType.DMA,
        ],
    )
    out = f(idx_all, input_embed, output_embed)
    return jnp.transpose(out, (1, 0, 2))
